# Initial kernel scaffold; baseline (speedup 1.0000x reference)
import jax
import jax.numpy as jnp
from jax.experimental import pallas as pl


def kernel(input, rois):
    raise NotImplementedError("write your pallas kernel here")


# speedup vs baseline: 5.3400x; 5.3400x over previous
import functools

import jax
import jax.numpy as jnp
from jax import lax
from jax.experimental import pallas as pl
from jax.experimental.pallas import tpu as pltpu
from jax.experimental.pallas import tpu_sc as plsc

N, C, H, W = 2, 256, 100, 100
R = 512
PH = PW = 7
SR = 2
SCALE = 0.25

NCORES = 2
NSUB = 16
NW = NCORES * NSUB
RPW = R // NW
NSAMP = PH * SR
NBIN = PH * PW
NCHUNK = 13
CV = C // 16


def _roi_align_body(flat, roisp, out, roi_v, ylw, yhw, xli, xhi,
                    lyf, hyf, lxf, hxf, wbuf, rows0, rows1, acc,
                    sem0, sem1):
    cid = lax.axis_index("c")
    sid = lax.axis_index("s")
    wid = sid * NCORES + cid
    roi_base = wid * RPW

    pltpu.sync_copy(roisp.at[wid], roi_v)

    io = lax.iota(jnp.int32, 16)
    iof = io.astype(jnp.float32)
    coeff = iof * 0.5 + 0.25
    lane_ok = io < NSAMP

    def axis_quantities(start, binsz, extent):
        ss = start + coeff * binsz
        valid = (ss >= -1.0) & (ss <= float(extent)) & lane_ok
        s0 = jnp.maximum(ss, 0.0)
        lo = jnp.minimum(s0.astype(jnp.int32), extent - 1)
        hi = jnp.minimum(lo + 1, extent - 1)
        frac = jnp.minimum(s0, float(extent - 1)) - lo.astype(jnp.float32)
        lofrac = jnp.where(valid, 1.0 - frac, 0.0)
        hifrac = jnp.where(valid, frac, 0.0)
        return lo, hi, lofrac, hifrac

    def splat(v):
        return jnp.full((16,), v, jnp.int32)

    def chunk_issue(q, b_v, rows_ref, wslot, sem):
        s = q * 16 + io
        bin_ = s >> 2
        rem = s & 3
        py = (bin_ * 9363) >> 16
        px = bin_ - py * 7
        ky = py * 2 + (rem >> 1)
        kx = px * 2 + (rem & 1)
        ylw_s = plsc.load_gather(ylw, [ky])
        yhw_s = plsc.load_gather(yhw, [ky])
        hy_s = plsc.load_gather(hyf, [ky])
        ly_s = plsc.load_gather(lyf, [ky])
        xl_s = plsc.load_gather(xli, [kx])
        xh_s = plsc.load_gather(xhi, [kx])
        hx_s = plsc.load_gather(hxf, [kx])
        lx_s = plsc.load_gather(lxf, [kx])
        wbuf[wslot, 0] = hy_s * hx_s
        wbuf[wslot, 1] = hy_s * lx_s
        wbuf[wslot, 2] = ly_s * hx_s
        wbuf[wslot, 3] = ly_s * lx_s
        i11 = b_v + ylw_s + xl_s
        i12 = b_v + ylw_s + xh_s
        i21 = b_v + yhw_s + xl_s
        i22 = b_v + yhw_s + xh_s
        pltpu.make_async_copy(flat.at[i11], rows_ref.at[pl.ds(0, 16)], sem).start()
        pltpu.make_async_copy(flat.at[i12], rows_ref.at[pl.ds(16, 16)], sem).start()
        pltpu.make_async_copy(flat.at[i21], rows_ref.at[pl.ds(32, 16)], sem).start()
        pltpu.make_async_copy(flat.at[i22], rows_ref.at[pl.ds(48, 16)], sem).start()

    def chunk_wait(rows_ref, sem):
        pltpu.make_async_copy(flat.at[pl.ds(0, 64)], rows_ref, sem).wait()

    def chunk_compute(q, rows_ref, wslot):
        def bin_body(t, _):
            accs = [jnp.zeros((16,), jnp.float32) for _ in range(CV)]
            for c2 in range(4):
                for i in range(4):
                    wsc = plsc.load_gather(
                        wbuf, [splat(wslot), splat(c2), splat(t * 4 + i)])
                    rb = c2 * 16 + t * 4 + i
                    for k in range(CV):
                        accs[k] = accs[k] + wsc * rows_ref[rb, pl.ds(k * 16, 16)]
            binrow = q * 4 + t
            for k in range(CV):
                acc[binrow, pl.ds(k * 16, 16)] = accs[k]
            return 0
        lax.fori_loop(0, 4, bin_body, 0)

    def roi_body(r, _):
        rsplat = splat(r)

        def field(i):
            return plsc.load_gather(roi_v, [splat(i), rsplat])

        b_v = field(0).astype(jnp.int32) * (H * W)
        sx_s = field(1) * SCALE - 0.5
        sy_s = field(2) * SCALE - 0.5
        ex_s = field(3) * SCALE - 0.5
        ey_s = field(4) * SCALE - 0.5
        bw_s = (ex_s - sx_s) / PW
        bh_s = (ey_s - sy_s) / PH
        yl_v, yh_v, hy_v, ly_v = axis_quantities(sy_s, bh_s, H)
        xl_v, xh_v, hx_v, lx_v = axis_quantities(sx_s, bw_s, W)
        ylw[...] = yl_v * W
        yhw[...] = yh_v * W
        xli[...] = xl_v
        xhi[...] = xh_v
        lyf[...] = ly_v
        hyf[...] = hy_v
        lxf[...] = lx_v * 0.25
        hxf[...] = hx_v * 0.25

        chunk_issue(0, b_v, rows0, 0, sem0)

        def pair_body(g, _):
            q = g * 2
            chunk_issue(q + 1, b_v, rows1, 1, sem1)
            chunk_wait(rows0, sem0)
            chunk_compute(q, rows0, 0)
            chunk_issue(q + 2, b_v, rows0, 0, sem0)
            chunk_wait(rows1, sem1)
            chunk_compute(q + 1, rows1, 1)
            return 0

        lax.fori_loop(0, (NCHUNK - 1) // 2, pair_body, 0)
        chunk_wait(rows0, sem0)
        chunk_compute(NCHUNK - 1, rows0, 0)
        pltpu.sync_copy(acc, out.at[roi_base + r])
        return 0

    lax.fori_loop(0, RPW, roi_body, 0)


@jax.jit
def _roi_align_sc(flat, roisp):
    mesh = plsc.VectorSubcoreMesh(core_axis_name="c", subcore_axis_name="s",
                                  num_cores=NCORES, num_subcores=NSUB)
    run = pl.kernel(
        _roi_align_body,
        out_type=jax.ShapeDtypeStruct((R, 52, C), jnp.float32),
        mesh=mesh,
        scratch_types=[
            pltpu.VMEM((5, RPW), jnp.float32),
            pltpu.VMEM((16,), jnp.int32),
            pltpu.VMEM((16,), jnp.int32),
            pltpu.VMEM((16,), jnp.int32),
            pltpu.VMEM((16,), jnp.int32),
            pltpu.VMEM((16,), jnp.float32),
            pltpu.VMEM((16,), jnp.float32),
            pltpu.VMEM((16,), jnp.float32),
            pltpu.VMEM((16,), jnp.float32),
            pltpu.VMEM((2, 4, 16), jnp.float32),
            pltpu.VMEM((64, C), jnp.float32),
            pltpu.VMEM((64, C), jnp.float32),
            pltpu.VMEM((52, C), jnp.float32),
            pltpu.SemaphoreType.DMA,
            pltpu.SemaphoreType.DMA,
        ],
        compiler_params=pltpu.CompilerParams(needs_layout_passes=False),
    )
    return run(flat, roisp)


def kernel(input, rois):
    flat = jnp.transpose(input, (0, 2, 3, 1)).reshape(N * H * W, C)
    roisp = jnp.transpose(rois, (1, 0)).reshape(5, NW, RPW).transpose(1, 0, 2)
    out = _roi_align_sc(flat, roisp)
    return out[:, :NBIN].reshape(R, PH, PW, C).transpose(0, 3, 1, 2)
